# baseline (speedup 1.0000x reference)
import functools

import jax
import jax.numpy as jnp
from jax import lax
from jax.experimental import pallas as pl
from jax.experimental.pallas import tpu as pltpu
from jax.experimental.pallas import tpu_sc as plsc

N = 10000
E = 320000
D = 128
H = 128
L = 4
C = 2
G = 64

NC, NS = 2, 16
NT = NC * NS
CW = 128
GC = 80
GC2 = GC // 2
EPT = GC * CW
ZR = 632
NPAD = ZR * NS
RB = 1000


def _sc_scatter_body(m_hbm, src_hbm, dst_hbm, zeros_hbm, out_hbm,
                     src_v, dst_v, gbuf_a, gbuf_b, acc, sem_a, sem_b):
    c = lax.axis_index("c")
    s = lax.axis_index("s")
    w = c * NS + s
    pltpu.sync_copy(src_hbm.at[w], src_v)
    pltpu.sync_copy(dst_hbm.at[w, pl.ds(0, GC2 * CW)], dst_v)
    pltpu.sync_copy(zeros_hbm, acc.at[pl.ds(s * ZR, ZR)])
    plsc.subcore_barrier()

    for phase in range(2):
        if phase == 1:
            pltpu.sync_copy(dst_hbm.at[w, pl.ds(GC2 * CW, GC2 * CW)], dst_v)
        base = phase * GC2

        for i in range(0, GC2, 2):
            j = base + i
            h_a = pltpu.async_copy(m_hbm.at[src_v.at[pl.ds(j * CW, CW)]],
                                   gbuf_a, sem_a)
            h_b = pltpu.async_copy(
                m_hbm.at[src_v.at[pl.ds((j + 1) * CW, CW)]], gbuf_b, sem_b)
            h_a.wait()
            pltpu.sync_copy(gbuf_a, acc.at[dst_v.at[pl.ds(i * CW, CW)]],
                            add=True)
            h_b.wait()
            pltpu.sync_copy(gbuf_b,
                            acc.at[dst_v.at[pl.ds((i + 1) * CW, CW)]],
                            add=True)
    plsc.subcore_barrier()
    pltpu.sync_copy(acc.at[pl.ds(s * ZR, ZR)], out_hbm.at[c, pl.ds(s * ZR, ZR)])


@functools.cache
def _get_sc_scatter():
    return pl.kernel(
        _sc_scatter_body,
        out_type=jax.ShapeDtypeStruct((NC, NPAD, H), jnp.float32),
        mesh=plsc.VectorSubcoreMesh(core_axis_name="c", subcore_axis_name="s",
                                    num_cores=NC, num_subcores=NS),
        scratch_types=[
            pltpu.VMEM((EPT,), jnp.int32),
            pltpu.VMEM((GC2 * CW,), jnp.int32),
            pltpu.VMEM((CW, H), jnp.float32),
            pltpu.VMEM((CW, H), jnp.float32),
            pltpu.VMEM_SHARED((NPAD, H), jnp.float32),
            pltpu.SemaphoreType.DMA,
            pltpu.SemaphoreType.DMA,
        ],
    )


def _tc_init_body(x_ref, w0_ref, b0_ref, wg0_ref, h_ref, m_ref):
    h = jnp.maximum(
        jnp.dot(x_ref[...], w0_ref[...], preferred_element_type=jnp.float32)
        + b0_ref[...], 0.0)
    h_ref[...] = h
    m_ref[...] = jnp.dot(h, wg0_ref[...], preferred_element_type=jnp.float32)


def _gru_compute(p_ref, h_ref, wihT_ref, whhT_ref, bih_ref, bhh_ref):
    agg = p_ref[0] + p_ref[1]
    gi = jnp.dot(agg, wihT_ref[...], preferred_element_type=jnp.float32) + bih_ref[...]
    gh = jnp.dot(h_ref[...], whhT_ref[...], preferred_element_type=jnp.float32) + bhh_ref[...]
    h0 = h_ref[...]
    r = jax.nn.sigmoid(gi[:, :H] + gh[:, :H])
    z = jax.nn.sigmoid(gi[:, H:2 * H] + gh[:, H:2 * H])
    n_ = jnp.tanh(gi[:, 2 * H:] + r * gh[:, 2 * H:])
    return (1.0 - z) * n_ + z * h0


def _tc_gru_body(p_ref, h_ref, wihT_ref, whhT_ref, bih_ref, bhh_ref, wgn_ref,
                 h_out, m_out):
    hn = _gru_compute(p_ref, h_ref, wihT_ref, whhT_ref, bih_ref, bhh_ref)
    h_out[...] = hn
    m_out[...] = jnp.dot(hn, wgn_ref[...], preferred_element_type=jnp.float32)


def _tc_gru_last_body(p_ref, h_ref, wihT_ref, whhT_ref, bih_ref, bhh_ref,
                      h_out):
    h_out[...] = _gru_compute(p_ref, h_ref, wihT_ref, whhT_ref, bih_ref, bhh_ref)


def _tc_head_body(h_ref, bat_ref, w1_ref, b1_ref, w2_ref, b2_ref, out_ref):
    bat = bat_ref[...]
    gids = lax.broadcasted_iota(jnp.int32, (G, N), 0)
    maskf = jnp.where(gids == bat, 1.0, 0.0)
    sums = jnp.dot(maskf, h_ref[...], preferred_element_type=jnp.float32)
    counts = jnp.sum(maskf, axis=1, keepdims=True)
    pooled = sums / jnp.maximum(counts, 1.0)
    a = jnp.maximum(
        jnp.dot(pooled, w1_ref[...], preferred_element_type=jnp.float32)
        + b1_ref[...], 0.0)
    logits = jnp.dot(a, w2_ref[...], preferred_element_type=jnp.float32) + b2_ref[...]
    mx = jnp.max(logits, axis=1, keepdims=True)
    lse = mx + jnp.log(jnp.sum(jnp.exp(logits - mx), axis=1, keepdims=True))
    out_ref[...] = logits - lse


_GRID = N // RB


def _row_spec(rows):
    return pl.BlockSpec((rows, 128), lambda i: (i, 0))


def _full_spec(shape):
    return pl.BlockSpec(shape, lambda i: (0,) * len(shape))


_tc_init = pl.pallas_call(
    _tc_init_body,
    grid=(_GRID,),
    in_specs=[_row_spec(RB), _full_spec((D, H)), _full_spec((1, H)),
              _full_spec((H, H))],
    out_specs=[_row_spec(RB), _row_spec(RB)],
    out_shape=[jax.ShapeDtypeStruct((N, H), jnp.float32),
               jax.ShapeDtypeStruct((N, H), jnp.float32)],
)

_p_spec = pl.BlockSpec((NC, RB, 128), lambda i: (0, i, 0))
_gru_common_specs = [_p_spec, _row_spec(RB), _full_spec((H, 3 * H)),
                     _full_spec((H, 3 * H)), _full_spec((1, 3 * H)),
                     _full_spec((1, 3 * H))]

_tc_gru = pl.pallas_call(
    _tc_gru_body,
    grid=(_GRID,),
    in_specs=_gru_common_specs + [_full_spec((H, H))],
    out_specs=[_row_spec(RB), _row_spec(RB)],
    out_shape=[jax.ShapeDtypeStruct((N, H), jnp.float32),
               jax.ShapeDtypeStruct((N, H), jnp.float32)],
)

_tc_gru_last = pl.pallas_call(
    _tc_gru_last_body,
    grid=(_GRID,),
    in_specs=_gru_common_specs,
    out_specs=[_row_spec(RB)],
    out_shape=[jax.ShapeDtypeStruct((N, H), jnp.float32)],
)

_tc_head = pl.pallas_call(
    _tc_head_body,
    out_shape=jax.ShapeDtypeStruct((G, H), jnp.float32),
)


def kernel(x, edge_index, batch, W0, b0, Wg, w_ih, w_hh, b_ih, b_hh, W1, b1,
           W2, b2):
    f32 = jnp.float32
    src = edge_index[0]
    dst = edge_index[1]
    pad = NT * EPT - E
    pad_idx = jnp.arange(pad, dtype=jnp.int32)
    src_t = jnp.concatenate([src, pad_idx % N]).reshape(NT, EPT)
    dst_t = jnp.concatenate([dst, N + pad_idx % (NPAD - N)]).reshape(NT, EPT)
    zeros_zr = jnp.zeros((ZR, H), f32)

    b0r = b0.reshape(1, H)
    bihr = b_ih.reshape(1, 3 * H)
    bhhr = b_hh.reshape(1, 3 * H)
    wihT = w_ih.T
    whhT = w_hh.T
    b1r = b1.reshape(1, H)
    w2p = jnp.zeros((H, H), f32).at[:, :C].set(W2)
    b2p = jnp.full((1, H), -1e30, f32).at[0, :C].set(b2)
    batr = batch.reshape(1, N)

    h, m = _tc_init(x, W0, b0r, Wg[0])
    for i in range(L):
        p = _get_sc_scatter()(m, src_t, dst_t, zeros_zr)
        if i + 1 < L:
            h, m = _tc_gru(p, h, wihT, whhT, bihr, bhhr, Wg[i + 1])
        else:
            (h,) = _tc_gru_last(p, h, wihT, whhT, bihr, bhhr)
    out = _tc_head(h, batr, W1, b1r, w2p, b2p)
    return out[:, :C]


# speedup vs baseline: 1.0160x; 1.0160x over previous
import functools

import jax
import jax.numpy as jnp
from jax import lax
from jax.experimental import pallas as pl
from jax.experimental.pallas import tpu as pltpu
from jax.experimental.pallas import tpu_sc as plsc

N = 10000
E = 320000
D = 128
H = 128
L = 4
C = 2
G = 64

NC, NS = 2, 16
NT = NC * NS
CW = 256
GC = 40
GC2 = GC // 2
EPT = GC * CW
ZR = 632
NPAD = ZR * NS
RB = 1000


def _sc_scatter_body(m_hbm, src_hbm, dst_hbm, zeros_hbm, out_hbm,
                     src_v, dst_v, gbuf_a, acc, sem_a):
    c = lax.axis_index("c")
    s = lax.axis_index("s")
    w = c * NS + s
    pltpu.sync_copy(src_hbm.at[w], src_v)
    pltpu.sync_copy(dst_hbm.at[w, pl.ds(0, GC2 * CW)], dst_v)
    pltpu.sync_copy(zeros_hbm, acc.at[pl.ds(s * ZR, ZR)])
    plsc.subcore_barrier()

    for phase in range(2):
        if phase == 1:
            pltpu.sync_copy(dst_hbm.at[w, pl.ds(GC2 * CW, GC2 * CW)], dst_v)
        base = phase * GC2

        def body(i, carry):
            j = base + i
            pltpu.sync_copy(m_hbm.at[src_v.at[pl.ds(j * CW, CW)]], gbuf_a)
            pltpu.sync_copy(gbuf_a, acc.at[dst_v.at[pl.ds(i * CW, CW)]],
                            add=True)
            return carry

        lax.fori_loop(0, GC2, body, 0)
    plsc.subcore_barrier()
    pltpu.sync_copy(acc.at[pl.ds(s * ZR, ZR)], out_hbm.at[c, pl.ds(s * ZR, ZR)])


@functools.cache
def _get_sc_scatter():
    return pl.kernel(
        _sc_scatter_body,
        out_type=jax.ShapeDtypeStruct((NC, NPAD, H), jnp.float32),
        mesh=plsc.VectorSubcoreMesh(core_axis_name="c", subcore_axis_name="s",
                                    num_cores=NC, num_subcores=NS),
        scratch_types=[
            pltpu.VMEM((EPT,), jnp.int32),
            pltpu.VMEM((GC2 * CW,), jnp.int32),
            pltpu.VMEM((CW, H), jnp.float32),
            pltpu.VMEM_SHARED((NPAD, H), jnp.float32),
            pltpu.SemaphoreType.DMA,
        ],
    )


def _tc_init_body(x_ref, w0_ref, b0_ref, wg0_ref, h_ref, m_ref):
    h = jnp.maximum(
        jnp.dot(x_ref[...], w0_ref[...], preferred_element_type=jnp.float32)
        + b0_ref[...], 0.0)
    h_ref[...] = h
    m_ref[...] = jnp.dot(h, wg0_ref[...], preferred_element_type=jnp.float32)


def _gru_compute(p_ref, h_ref, wihT_ref, whhT_ref, bih_ref, bhh_ref):
    agg = p_ref[0] + p_ref[1]
    gi = jnp.dot(agg, wihT_ref[...], preferred_element_type=jnp.float32) + bih_ref[...]
    gh = jnp.dot(h_ref[...], whhT_ref[...], preferred_element_type=jnp.float32) + bhh_ref[...]
    h0 = h_ref[...]
    r = jax.nn.sigmoid(gi[:, :H] + gh[:, :H])
    z = jax.nn.sigmoid(gi[:, H:2 * H] + gh[:, H:2 * H])
    n_ = jnp.tanh(gi[:, 2 * H:] + r * gh[:, 2 * H:])
    return (1.0 - z) * n_ + z * h0


def _tc_gru_body(p_ref, h_ref, wihT_ref, whhT_ref, bih_ref, bhh_ref, wgn_ref,
                 h_out, m_out):
    hn = _gru_compute(p_ref, h_ref, wihT_ref, whhT_ref, bih_ref, bhh_ref)
    h_out[...] = hn
    m_out[...] = jnp.dot(hn, wgn_ref[...], preferred_element_type=jnp.float32)


def _tc_tail_body(p_ref, h_ref, wihT_ref, whhT_ref, bih_ref, bhh_ref,
                  bat_ref, w1_ref, b1_ref, w2_ref, b2_ref, out_ref,
                  pool_acc, cnt_acc):
    i = pl.program_id(0)
    hn = _gru_compute(p_ref, h_ref, wihT_ref, whhT_ref, bih_ref, bhh_ref)
    bat = bat_ref[0]
    gids = lax.broadcasted_iota(jnp.int32, (G, RB), 0)
    maskf = jnp.where(gids == bat, 1.0, 0.0)
    part = jnp.dot(maskf, hn, preferred_element_type=jnp.float32)
    cnt = jnp.sum(maskf, axis=1, keepdims=True)

    @pl.when(i == 0)
    def _init():
        pool_acc[...] = jnp.zeros_like(pool_acc)
        cnt_acc[...] = jnp.zeros_like(cnt_acc)

    pool_acc[...] += part
    cnt_acc[...] = cnt_acc[...] + cnt

    @pl.when(i == _GRID - 1)
    def _head():
        pooled = pool_acc[...] / jnp.maximum(cnt_acc[...], 1.0)
        a = jnp.maximum(
            jnp.dot(pooled, w1_ref[...], preferred_element_type=jnp.float32)
            + b1_ref[...], 0.0)
        logits = jnp.dot(a, w2_ref[...],
                         preferred_element_type=jnp.float32) + b2_ref[...]
        mx = jnp.max(logits, axis=1, keepdims=True)
        lse = mx + jnp.log(jnp.sum(jnp.exp(logits - mx), axis=1,
                                   keepdims=True))
        out_ref[...] = logits - lse


_GRID = N // RB


def _row_spec(rows):
    return pl.BlockSpec((rows, 128), lambda i: (i, 0))


def _full_spec(shape):
    return pl.BlockSpec(shape, lambda i: (0,) * len(shape))


_tc_init = pl.pallas_call(
    _tc_init_body,
    grid=(_GRID,),
    in_specs=[_row_spec(RB), _full_spec((D, H)), _full_spec((1, H)),
              _full_spec((H, H))],
    out_specs=[_row_spec(RB), _row_spec(RB)],
    out_shape=[jax.ShapeDtypeStruct((N, H), jnp.float32),
               jax.ShapeDtypeStruct((N, H), jnp.float32)],
)

_p_spec = pl.BlockSpec((NC, RB, 128), lambda i: (0, i, 0))
_gru_common_specs = [_p_spec, _row_spec(RB), _full_spec((H, 3 * H)),
                     _full_spec((H, 3 * H)), _full_spec((1, 3 * H)),
                     _full_spec((1, 3 * H))]

_tc_gru = pl.pallas_call(
    _tc_gru_body,
    grid=(_GRID,),
    in_specs=_gru_common_specs + [_full_spec((H, H))],
    out_specs=[_row_spec(RB), _row_spec(RB)],
    out_shape=[jax.ShapeDtypeStruct((N, H), jnp.float32),
               jax.ShapeDtypeStruct((N, H), jnp.float32)],
)

_tc_tail = pl.pallas_call(
    _tc_tail_body,
    grid=(_GRID,),
    in_specs=_gru_common_specs + [
        pl.BlockSpec((1, 1, RB), lambda i: (i, 0, 0)),
        _full_spec((H, H)), _full_spec((1, H)),
        _full_spec((H, H)), _full_spec((1, H)),
    ],
    out_specs=[pl.BlockSpec((G, H), lambda i: (0, 0))],
    out_shape=[jax.ShapeDtypeStruct((G, H), jnp.float32)],
    scratch_shapes=[pltpu.VMEM((G, H), jnp.float32),
                    pltpu.VMEM((G, 1), jnp.float32)],
)


def kernel(x, edge_index, batch, W0, b0, Wg, w_ih, w_hh, b_ih, b_hh, W1, b1,
           W2, b2):
    f32 = jnp.float32
    src = edge_index[0]
    dst = edge_index[1]
    pad = NT * EPT - E
    pad_idx = jnp.arange(pad, dtype=jnp.int32)
    src_t = jnp.concatenate([src, pad_idx % N]).reshape(NT, EPT)
    dst_t = jnp.concatenate([dst, N + pad_idx % (NPAD - N)]).reshape(NT, EPT)
    zeros_zr = jnp.zeros((ZR, H), f32)

    b0r = b0.reshape(1, H)
    bihr = b_ih.reshape(1, 3 * H)
    bhhr = b_hh.reshape(1, 3 * H)
    wihT = w_ih.T
    whhT = w_hh.T
    b1r = b1.reshape(1, H)
    w2p = jnp.zeros((H, H), f32).at[:, :C].set(W2)
    b2p = jnp.full((1, H), -1e30, f32).at[0, :C].set(b2)
    batr = batch.reshape(_GRID, 1, RB)

    h, m = _tc_init(x, W0, b0r, Wg[0])
    for i in range(L):
        p = _get_sc_scatter()(m, src_t, dst_t, zeros_zr)
        if i + 1 < L:
            h, m = _tc_gru(p, h, wihT, whhT, bihr, bhhr, Wg[i + 1])
        else:
            (out,) = _tc_tail(p, h, wihT, whhT, bihr, bhhr, batr, W1, b1r,
                              w2p, b2p)
    return out[:, :C]
